# baseline (speedup 1.0000x reference)
import jax
import jax.numpy as jnp
from jax import lax
from jax.experimental import pallas as pl
from jax.experimental.pallas import tpu as pltpu
from jax.experimental.pallas import tpu_sc as plsc

N_FIELDS = 26
VOCAB = 100000
D_MODEL = 64
BATCH = 16384

NC, NS, L = 2, 16, 16
NW = NC * NS
CHUNK = BATCH // NW
IDX_W = 128
GPC = CHUNK // IDX_W

_mesh = plsc.VectorSubcoreMesh(core_axis_name="c", subcore_axis_name="s")


def _prep_body(xn_hbm, x1_hbm, vrow):
    wid = lax.axis_index("s") * NC + lax.axis_index("c")
    b0 = wid * CHUNK
    for f in range(N_FIELDS):
        pltpu.sync_copy(xn_hbm.at[f, pl.ds(b0, CHUNK)], vrow)
        off = f * VOCAB
        for k in range(CHUNK // L):
            vrow[pl.ds(k * L, L)] = vrow[pl.ds(k * L, L)] + off
        pltpu.sync_copy(vrow, x1_hbm.at[pl.ds(f * BATCH + b0, CHUNK)])


_prep = pl.kernel(
    _prep_body,
    out_type=jax.ShapeDtypeStruct((N_FIELDS * BATCH,), jnp.int32),
    mesh=_mesh,
    scratch_types=[pltpu.VMEM((CHUNK,), jnp.int32)],
    compiler_params=pltpu.CompilerParams(use_tc_tiling_on_sc=True),
)

ROWS = N_FIELDS * VOCAB
TW = 384
WPF = VOCAB // TW
TREM = VOCAB - WPF * TW
NT = N_FIELDS * WPF
KPW = NT // NW


def _transpose_body(tabt_hbm, tail_hbm, tabl_hbm, vin0, vin1, vout0, vout1,
                    vtail, rs0, rs1, ws0, ws1):
    wid = lax.axis_index("s") * NC + lax.axis_index("c")
    vins = (vin0, vin1)
    vouts = (vout0, vout1)
    rsems = (rs0, rs1)
    wsems = (ws0, ws1)
    lanes = lax.iota(jnp.int32, L)

    def read(t, b):
        f = t // WPF
        v0 = pl.multiple_of((t % WPF) * TW, TW)
        pltpu.async_copy(tabt_hbm.at[f, :, pl.ds(v0, TW)], vins[b], rsems[b])

    def wait_read(b):
        pltpu.make_async_copy(tabt_hbm.at[0, :, pl.ds(0, TW)], vins[b],
                              rsems[b]).wait()

    def xpose(b):
        def row(j, _):
            for h in range(2):
                vcol = jnp.full((L,), 2 * j + h, jnp.int32)
                for k in range(D_MODEL // L):
                    g = plsc.load_gather(vins[b], [lanes + k * L, vcol])
                    vouts[b][j, pl.ds(h * D_MODEL + k * L, L)] = g
            return 0
        lax.fori_loop(0, TW // 2, row, 0)

    def write(t, b):
        f = t // WPF
        u0 = pl.multiple_of(f * (VOCAB // 2) + (t % WPF) * (TW // 2), 8)
        pltpu.async_copy(vouts[b], tabl_hbm.at[pl.ds(u0, TW // 2), :],
                         wsems[b])

    def wait_write(b):
        pltpu.make_async_copy(tabl_hbm.at[pl.ds(0, TW // 2), :], vouts[b],
                              wsems[b]).wait()

    read(wid, 0)

    def pair(p, _):
        for b in range(2):
            k = 2 * p + b
            t = wid + NW * k
            wait_read(b)

            @pl.when(k + 1 < KPW)
            def _():
                read(wid + NW * (k + 1), 1 - b)

            @pl.when(k >= 2)
            def _():
                wait_write(b)
            xpose(b)
            write(t, b)
        return 0

    lax.fori_loop(0, KPW // 2, pair, 0)
    wait_read(0)
    wait_write(0)
    xpose(0)
    write(wid + NW * (KPW - 1), 0)
    wait_write(1)
    wait_write(0)

    @pl.when(wid < NT - NW * KPW)
    def _():
        t = NW * KPW + wid
        f = t // WPF
        v0 = pl.multiple_of((t % WPF) * TW, TW)
        pltpu.sync_copy(tabt_hbm.at[f, :, pl.ds(v0, TW)], vin0)
        xpose(0)
        u0 = pl.multiple_of(f * (VOCAB // 2) + (t % WPF) * (TW // 2), 8)
        pltpu.sync_copy(vout0, tabl_hbm.at[pl.ds(u0, TW // 2), :])

    @pl.when(wid < N_FIELDS)
    def _():
        f = wid
        pltpu.sync_copy(tail_hbm.at[f], vtail)

        def row(j, _):
            for h in range(2):
                for k in range(D_MODEL // L):
                    vout0[j, pl.ds(h * D_MODEL + k * L, L)] = (
                        vtail[2 * j + h, pl.ds(k * L, L)])
            return 0
        lax.fori_loop(0, TREM // 2, row, 0)
        u0 = pl.multiple_of(f * (VOCAB // 2) + WPF * (TW // 2), 8)
        pltpu.sync_copy(vout0.at[pl.ds(0, TREM // 2), :],
                        tabl_hbm.at[pl.ds(u0, TREM // 2), :])


_transpose = pl.kernel(
    _transpose_body,
    out_type=jax.ShapeDtypeStruct((ROWS // 2, 128), jnp.float32),
    mesh=_mesh,
    scratch_types=[
        pltpu.VMEM((D_MODEL, TW), jnp.float32),
        pltpu.VMEM((D_MODEL, TW), jnp.float32),
        pltpu.VMEM((TW // 2, 128), jnp.float32),
        pltpu.VMEM((TW // 2, 128), jnp.float32),
        pltpu.VMEM((TREM, 128), jnp.float32),
        pltpu.SemaphoreType.DMA,
        pltpu.SemaphoreType.DMA,
        pltpu.SemaphoreType.DMA,
        pltpu.SemaphoreType.DMA,
    ],
    compiler_params=pltpu.CompilerParams(use_tc_tiling_on_sc=True,
                                         needs_layout_passes=False),
)


def _gather_body(x1_hbm, tab_hbm, out_hbm, idx0, idx1, rows0, rows1,
                 gs0, gs1, ws0, ws1):
    wid = lax.axis_index("s") * NC + lax.axis_index("c")
    b0 = wid * CHUNK

    idxs = (idx0, idx1)
    bufs = (rows0, rows1)
    gsems = (gs0, gs1)
    wsems = (ws0, ws1)

    def load_idx(f, b):
        for q in range(GPC):
            pltpu.sync_copy(
                x1_hbm.at[pl.ds(f * BATCH + b0 + q * IDX_W, IDX_W)],
                idxs[b].at[q])

    def fire(b):
        for q in range(GPC):
            pltpu.async_copy(
                tab_hbm.at[idxs[b].at[q]],
                bufs[b].at[pl.ds(q * IDX_W, IDX_W)],
                gsems[b])

    def wait_full(b, sem):
        pltpu.make_async_copy(out_hbm.at[0, pl.ds(0, CHUNK)], bufs[b], sem).wait()

    def put(f, b):
        pltpu.async_copy(bufs[b], out_hbm.at[f, pl.ds(b0, CHUNK)], wsems[b])

    load_idx(0, 0)
    fire(0)

    def pair(p, _):
        f0 = 2 * p

        @pl.when(p >= 1)
        def _():
            wait_full(1, wsems[1])
        load_idx(f0 + 1, 1)
        fire(1)

        wait_full(0, gsems[0])
        put(f0, 0)
        wait_full(0, wsems[0])

        @pl.when(p < N_FIELDS // 2 - 1)
        def _():
            load_idx(f0 + 2, 0)
            fire(0)

        wait_full(1, gsems[1])
        put(f0 + 1, 1)
        return 0

    lax.fori_loop(0, N_FIELDS // 2, pair, 0)
    wait_full(1, wsems[1])


_gather = pl.kernel(
    _gather_body,
    out_type=jax.ShapeDtypeStruct((N_FIELDS, BATCH, D_MODEL), jnp.float32),
    mesh=_mesh,
    scratch_types=[
        pltpu.VMEM((GPC, IDX_W), jnp.int32),
        pltpu.VMEM((GPC, IDX_W), jnp.int32),
        pltpu.VMEM((CHUNK, D_MODEL), jnp.float32),
        pltpu.VMEM((CHUNK, D_MODEL), jnp.float32),
        pltpu.SemaphoreType.DMA,
        pltpu.SemaphoreType.DMA,
        pltpu.SemaphoreType.DMA,
        pltpu.SemaphoreType.DMA,
    ],
    compiler_params=pltpu.CompilerParams(use_tc_tiling_on_sc=False),
)


@jax.jit
def kernel(x_cat, tables):
    x1 = _prep(x_cat.T)
    tail = jnp.pad(tables[:, WPF * TW:, :], ((0, 0), (0, 0), (0, 128 - D_MODEL)))
    tabl = _transpose(tables.transpose(0, 2, 1), tail)
    out = _gather(x1, tabl.reshape(ROWS, D_MODEL))
    return out.transpose(1, 0, 2)


# speedup vs baseline: 2.5323x; 2.5323x over previous
import jax
import jax.numpy as jnp
from jax import lax
from jax.experimental import pallas as pl
from jax.experimental.pallas import tpu as pltpu
from jax.experimental.pallas import tpu_sc as plsc

N_FIELDS = 26
VOCAB = 100000
D_MODEL = 64
BATCH = 16384
ROWS = N_FIELDS * VOCAB

NC, NS, L = 2, 16, 16
NW = NC * NS
WIN = BATCH // NW
CHUNK = 256
IDX_W = 128
GPC = CHUNK // IDX_W
STEPS = N_FIELDS * (WIN // CHUNK)

_mesh = plsc.VectorSubcoreMesh(core_axis_name="c", subcore_axis_name="s")


def _prep_body(xn_hbm, x1_hbm, vrow):
    wid = lax.axis_index("s") * NC + lax.axis_index("c")
    b0 = wid * WIN
    for f in range(N_FIELDS):
        pltpu.sync_copy(xn_hbm.at[f, pl.ds(b0, WIN)], vrow)
        off = f * VOCAB
        for k in range(WIN // L):
            vrow[pl.ds(k * L, L)] = vrow[pl.ds(k * L, L)] + off
        pltpu.sync_copy(vrow, x1_hbm.at[pl.ds(f * BATCH + b0, WIN)])


_prep = pl.kernel(
    _prep_body,
    out_type=jax.ShapeDtypeStruct((N_FIELDS * BATCH,), jnp.int32),
    mesh=_mesh,
    scratch_types=[pltpu.VMEM((WIN,), jnp.int32)],
    compiler_params=pltpu.CompilerParams(use_tc_tiling_on_sc=True),
)


def _gather_body(x1_hbm, tab_hbm, out_hbm, idx0, idx1, rows0, rows1,
                 gs0, gs1, ws0, ws1):
    wid = lax.axis_index("s") * NC + lax.axis_index("c")
    b0 = wid * WIN

    idxs = (idx0, idx1)
    bufs = (rows0, rows1)
    gsems = (gs0, gs1)
    wsems = (ws0, ws1)

    def load_idx(s, b):
        off = pl.multiple_of((s // 2) * BATCH + b0 + (s % 2) * CHUNK, IDX_W)
        for q in range(GPC):
            pltpu.sync_copy(x1_hbm.at[pl.ds(off + q * IDX_W, IDX_W)],
                            idxs[b].at[q])

    def fire(b):
        for q in range(GPC):
            pltpu.async_copy(
                tab_hbm.at[idxs[b].at[q]],
                bufs[b].at[pl.ds(q * IDX_W, IDX_W), :],
                gsems[b])

    def wait_gather(b):
        pltpu.make_async_copy(tab_hbm.at[pl.ds(0, CHUNK), :], bufs[b],
                              gsems[b]).wait()

    def put(s, b):
        pltpu.async_copy(
            bufs[b].at[:, pl.ds(0, D_MODEL)],
            out_hbm.at[s // 2, pl.ds(b0 + (s % 2) * CHUNK, CHUNK), :],
            wsems[b])

    def wait_put(b):
        pltpu.make_async_copy(out_hbm.at[0, pl.ds(0, CHUNK), :],
                              bufs[b].at[:, pl.ds(0, D_MODEL)],
                              wsems[b]).wait()

    load_idx(0, 0)
    fire(0)

    def pair(p, _):
        s0 = 2 * p

        @pl.when(p >= 1)
        def _():
            wait_put(1)
        load_idx(s0 + 1, 1)
        fire(1)

        wait_gather(0)
        put(s0, 0)
        wait_put(0)

        @pl.when(p < STEPS // 2 - 1)
        def _():
            load_idx(s0 + 2, 0)
            fire(0)

        wait_gather(1)
        put(s0 + 1, 1)
        return 0

    lax.fori_loop(0, STEPS // 2, pair, 0)
    wait_put(1)


_gather = pl.kernel(
    _gather_body,
    out_type=jax.ShapeDtypeStruct((N_FIELDS, BATCH, D_MODEL), jnp.float32),
    mesh=_mesh,
    scratch_types=[
        pltpu.VMEM((GPC, IDX_W), jnp.int32),
        pltpu.VMEM((GPC, IDX_W), jnp.int32),
        pltpu.VMEM((CHUNK, 2 * D_MODEL), jnp.float32),
        pltpu.VMEM((CHUNK, 2 * D_MODEL), jnp.float32),
        pltpu.SemaphoreType.DMA,
        pltpu.SemaphoreType.DMA,
        pltpu.SemaphoreType.DMA,
        pltpu.SemaphoreType.DMA,
    ],
    compiler_params=pltpu.CompilerParams(use_tc_tiling_on_sc=False),
)


@jax.jit
def kernel(x_cat, tables):
    x1 = _prep(x_cat.T)
    tabp = jnp.pad(tables, ((0, 0), (0, 0), (0, 128 - D_MODEL)))
    out = _gather(x1, tabp.reshape(ROWS, 2 * D_MODEL))
    return out.transpose(1, 0, 2)


# speedup vs baseline: 2.5984x; 1.0261x over previous
import jax
import jax.numpy as jnp
from jax import lax
from jax.experimental import pallas as pl
from jax.experimental.pallas import tpu as pltpu
from jax.experimental.pallas import tpu_sc as plsc

N_FIELDS = 26
VOCAB = 100000
D_MODEL = 64
BATCH = 16384
ROWS = N_FIELDS * VOCAB

NC, NS, L = 2, 16, 16
NW = NC * NS
WIN = BATCH // NW
CHUNK = 256
IDX_W = 128
GPC = CHUNK // IDX_W
STEPS = N_FIELDS * (WIN // CHUNK)

_mesh = plsc.VectorSubcoreMesh(core_axis_name="c", subcore_axis_name="s")


def _prep_body(xn_hbm, x1_hbm, vrow):
    wid = lax.axis_index("s") * NC + lax.axis_index("c")
    b0 = wid * WIN
    for f in range(N_FIELDS):
        pltpu.sync_copy(xn_hbm.at[f, pl.ds(b0, WIN)], vrow)
        off = f * VOCAB
        for k in range(WIN // L):
            vrow[pl.ds(k * L, L)] = (vrow[pl.ds(k * L, L)] + off) * 2
        pltpu.sync_copy(vrow, x1_hbm.at[pl.ds(f * BATCH + b0, WIN)])


_prep = pl.kernel(
    _prep_body,
    out_type=jax.ShapeDtypeStruct((N_FIELDS * BATCH,), jnp.int32),
    mesh=_mesh,
    scratch_types=[pltpu.VMEM((WIN,), jnp.int32)],
    compiler_params=pltpu.CompilerParams(use_tc_tiling_on_sc=True),
)


def _gather_body(x1_hbm, tab_hbm, out_hbm, idx0, idx1, rows0, rows1,
                 gs0, gs1, ws0, ws1):
    wid = lax.axis_index("s") * NC + lax.axis_index("c")
    b0 = wid * WIN

    idxs = (idx0, idx1)
    bufs = (rows0, rows1)
    gsems = (gs0, gs1)
    wsems = (ws0, ws1)

    def load_idx(s, b):
        off = pl.multiple_of((s // 2) * BATCH + b0 + (s % 2) * CHUNK, IDX_W)
        for q in range(GPC):
            pltpu.sync_copy(x1_hbm.at[pl.ds(off + q * IDX_W, IDX_W)],
                            idxs[b].at[q])

    def fire(b):
        for q in range(GPC):
            pltpu.async_copy(
                tab_hbm.at[idxs[b].at[q]],
                bufs[b].at[pl.ds(q * IDX_W, IDX_W), :],
                gsems[b])

    def wait_gather(b):
        pltpu.make_async_copy(tab_hbm.at[pl.ds(0, CHUNK), :], bufs[b],
                              gsems[b]).wait()

    def put(s, b):
        pltpu.async_copy(
            bufs[b],
            out_hbm.at[s // 2, pl.ds(b0 + (s % 2) * CHUNK, CHUNK), :],
            wsems[b])

    def wait_put(b):
        pltpu.make_async_copy(out_hbm.at[0, pl.ds(0, CHUNK), :], bufs[b],
                              wsems[b]).wait()

    load_idx(0, 0)
    fire(0)

    def pair(p, _):
        s0 = 2 * p

        @pl.when(p >= 1)
        def _():
            wait_put(1)
        load_idx(s0 + 1, 1)
        fire(1)

        wait_gather(0)
        put(s0, 0)
        wait_put(0)

        @pl.when(p < STEPS // 2 - 1)
        def _():
            load_idx(s0 + 2, 0)
            fire(0)

        wait_gather(1)
        put(s0 + 1, 1)
        return 0

    lax.fori_loop(0, STEPS // 2, pair, 0)
    wait_put(1)


_gather = pl.kernel(
    _gather_body,
    out_type=jax.ShapeDtypeStruct((N_FIELDS, BATCH, D_MODEL), jnp.float32),
    mesh=_mesh,
    scratch_types=[
        pltpu.VMEM((GPC, IDX_W), jnp.int32),
        pltpu.VMEM((GPC, IDX_W), jnp.int32),
        pltpu.VMEM((CHUNK, D_MODEL), jnp.float32),
        pltpu.VMEM((CHUNK, D_MODEL), jnp.float32),
        pltpu.SemaphoreType.DMA,
        pltpu.SemaphoreType.DMA,
        pltpu.SemaphoreType.DMA,
        pltpu.SemaphoreType.DMA,
    ],
    compiler_params=pltpu.CompilerParams(use_tc_tiling_on_sc=False),
)


@jax.jit
def kernel(x_cat, tables):
    x1 = _prep(x_cat.T)
    tabp = jnp.pad(tables, ((0, 0), (0, 0), (0, 128 - D_MODEL)))
    out = _gather(x1, tabp.reshape(2 * ROWS, D_MODEL))
    return out.transpose(1, 0, 2)
